# baseline (device time: 61724 ns/iter reference)
import functools

import jax
import jax.numpy as jnp
from jax import lax
from jax.experimental import pallas as pl
from jax.experimental.pallas import tpu as pltpu

N_DEV = 8
N_TOK = 512
D_IN = 256
D_OUT = 512
N_EXP = 16
EXP_PER_DEV = N_EXP // N_DEV


def kernel(x, router_W, route_idx, expert_W):
    def body(x_ref, rw_ref, idx_ref, ew_ref, out_ref, comm_ref, send_sems, recv_sems):
        my = lax.axis_index("i")
        left = lax.rem(my + N_DEV - 1, N_DEV)
        right = lax.rem(my + 1, N_DEV)

        barrier_sem = pltpu.get_barrier_semaphore()
        for nbr in (left, right):
            pl.semaphore_signal(
                barrier_sem, inc=1,
                device_id=(nbr,), device_id_type=pl.DeviceIdType.MESH,
            )
        pl.semaphore_wait(barrier_sem, 2)

        xf = x_ref[:, :]
        scores = jnp.dot(xf, rw_ref[:, :], preferred_element_type=jnp.float32)
        s_max = jnp.max(scores, axis=-1, keepdims=True)
        ex = jnp.exp(scores - s_max)
        probs = ex / jnp.sum(ex, axis=-1, keepdims=True)

        top0 = idx_ref[:, 0:1]
        top1 = idx_ref[:, 1:2]
        iota = lax.broadcasted_iota(jnp.int32, (N_TOK, N_EXP), 1)
        p0 = jnp.sum(jnp.where(iota == top0, probs, 0.0), axis=-1, keepdims=True)
        p1 = jnp.sum(jnp.where(iota == top1, probs, 0.0), axis=-1, keepdims=True)
        gsum = p0 + p1

        xb = xf.astype(jnp.bfloat16)
        acc = jnp.zeros((N_TOK, D_OUT), jnp.float32)
        for j in range(EXP_PER_DEV):
            g = my * EXP_PER_DEV + j
            routed = jnp.logical_or(top0 == g, top1 == g)
            pg = jnp.sum(jnp.where(iota == g, probs, 0.0), axis=-1, keepdims=True)
            w = jnp.where(routed, pg / gsum, 0.0)
            y = jnp.dot(
                xb, ew_ref[j, :, :].astype(jnp.bfloat16),
                preferred_element_type=jnp.float32,
            )
            acc = acc + w * y

        out_ref[:, :] = acc
        comm_ref[0, :, :] = acc.astype(jnp.bfloat16)

        for h in range(N_DEV - 1):
            rdma = pltpu.make_async_remote_copy(
                src_ref=comm_ref.at[h],
                dst_ref=comm_ref.at[h + 1],
                send_sem=send_sems.at[h],
                recv_sem=recv_sems.at[h],
                device_id=(right,),
                device_id_type=pl.DeviceIdType.MESH,
            )
            rdma.start()
            rdma.wait()
            out_ref[:, :] = out_ref[:, :] + comm_ref[h + 1, :, :].astype(jnp.float32)

    return pl.pallas_call(
        body,
        out_shape=jax.ShapeDtypeStruct((N_TOK, D_OUT), jnp.float32),
        in_specs=[
            pl.BlockSpec(memory_space=pltpu.VMEM),
            pl.BlockSpec(memory_space=pltpu.VMEM),
            pl.BlockSpec(memory_space=pltpu.VMEM),
            pl.BlockSpec(memory_space=pltpu.VMEM),
        ],
        out_specs=pl.BlockSpec(memory_space=pltpu.VMEM),
        scratch_shapes=[
            pltpu.VMEM((N_DEV, N_TOK, D_OUT), jnp.bfloat16),
            pltpu.SemaphoreType.DMA((N_DEV - 1,)),
            pltpu.SemaphoreType.DMA((N_DEV - 1,)),
        ],
        compiler_params=pltpu.CompilerParams(collective_id=0),
    )(x, router_W, route_idx, expert_W)


# device time: 31168 ns/iter; 1.9804x vs baseline; 1.9804x over previous
import functools

import jax
import jax.numpy as jnp
from jax import lax
from jax.experimental import pallas as pl
from jax.experimental.pallas import tpu as pltpu

N_DEV = 8
N_TOK = 512
D_IN = 256
D_OUT = 512
N_EXP = 16
EXP_PER_DEV = N_EXP // N_DEV


def kernel(x, router_W, route_idx, expert_W):
    XOR_MASKS = (1, 3, 4)

    def body(x_ref, rw_ref, idx_ref, ew_ref, out_ref, sendbuf, recvbuf, send_sems, recv_sems):
        my = lax.axis_index("i")
        partners = [jnp.bitwise_xor(my, m) for m in XOR_MASKS]

        barrier_sem = pltpu.get_barrier_semaphore()
        for nbr in partners:
            pl.semaphore_signal(
                barrier_sem, inc=1,
                device_id=(nbr,), device_id_type=pl.DeviceIdType.MESH,
            )
        pl.semaphore_wait(barrier_sem, len(partners))

        xf = x_ref[:, :]
        scores = jnp.dot(xf, rw_ref[:, :], preferred_element_type=jnp.float32)
        s_max = jnp.max(scores, axis=-1, keepdims=True)
        ex = jnp.exp(scores - s_max)
        probs = ex / jnp.sum(ex, axis=-1, keepdims=True)

        top0 = idx_ref[:, 0:1]
        top1 = idx_ref[:, 1:2]
        iota = lax.broadcasted_iota(jnp.int32, (N_TOK, N_EXP), 1)
        p0 = jnp.sum(jnp.where(iota == top0, probs, 0.0), axis=-1, keepdims=True)
        p1 = jnp.sum(jnp.where(iota == top1, probs, 0.0), axis=-1, keepdims=True)
        gsum = p0 + p1

        xb = xf.astype(jnp.bfloat16)
        acc = jnp.zeros((N_TOK, D_OUT), jnp.float32)
        for j in range(EXP_PER_DEV):
            g = my * EXP_PER_DEV + j
            routed = jnp.logical_or(top0 == g, top1 == g)
            pg = jnp.sum(jnp.where(iota == g, probs, 0.0), axis=-1, keepdims=True)
            w = jnp.where(routed, pg / gsum, 0.0)
            y = jnp.dot(
                xb, ew_ref[j, :, :].astype(jnp.bfloat16),
                preferred_element_type=jnp.float32,
            )
            acc = acc + w * y

        cur = acc
        for r in range(3):
            sendbuf[r, :, :] = cur.astype(jnp.bfloat16)
            rdma = pltpu.make_async_remote_copy(
                src_ref=sendbuf.at[r],
                dst_ref=recvbuf.at[r],
                send_sem=send_sems.at[r],
                recv_sem=recv_sems.at[r],
                device_id=(partners[r],),
                device_id_type=pl.DeviceIdType.MESH,
            )
            rdma.start()
            rdma.wait()
            cur = cur + recvbuf[r, :, :].astype(jnp.float32)
        out_ref[:, :] = cur

    return pl.pallas_call(
        body,
        out_shape=jax.ShapeDtypeStruct((N_TOK, D_OUT), jnp.float32),
        in_specs=[
            pl.BlockSpec(memory_space=pltpu.VMEM),
            pl.BlockSpec(memory_space=pltpu.VMEM),
            pl.BlockSpec(memory_space=pltpu.VMEM),
            pl.BlockSpec(memory_space=pltpu.VMEM),
        ],
        out_specs=pl.BlockSpec(memory_space=pltpu.VMEM),
        scratch_shapes=[
            pltpu.VMEM((3, N_TOK, D_OUT), jnp.bfloat16),
            pltpu.VMEM((3, N_TOK, D_OUT), jnp.bfloat16),
            pltpu.SemaphoreType.DMA((3,)),
            pltpu.SemaphoreType.DMA((3,)),
        ],
        compiler_params=pltpu.CompilerParams(collective_id=0),
    )(x, router_W, route_idx, expert_W)


# device time: 20513 ns/iter; 3.0090x vs baseline; 1.5194x over previous
import functools

import jax
import jax.numpy as jnp
from jax import lax
from jax.experimental import pallas as pl
from jax.experimental.pallas import tpu as pltpu

N_DEV = 8
N_TOK = 512
D_IN = 256
D_OUT = 512
N_EXP = 16
EXP_PER_DEV = N_EXP // N_DEV


def kernel(x, router_W, route_idx, expert_W):
    XOR_MASKS = (1, 3, 4)
    STRIPES = ((0, 176), (176, 344), (344, 512))

    def body(x_ref, rw_ref, idx_ref, ew_ref, out_ref,
             sb0, sb1, sb2, rb0, rb1, rb2, send_sems, recv_sems):
        my = lax.axis_index("i")
        partners = [jnp.bitwise_xor(my, m) for m in XOR_MASKS]

        barrier_sem = pltpu.get_barrier_semaphore()
        for nbr in partners:
            pl.semaphore_signal(
                barrier_sem, inc=1,
                device_id=(nbr,), device_id_type=pl.DeviceIdType.MESH,
            )
        pl.semaphore_wait(barrier_sem, len(partners))

        xf = x_ref[:, :]
        scores = jnp.dot(xf, rw_ref[:, :], preferred_element_type=jnp.float32)
        s_max = jnp.max(scores, axis=-1, keepdims=True)
        ex = jnp.exp(scores - s_max)
        probs = ex / jnp.sum(ex, axis=-1, keepdims=True)

        top0 = idx_ref[:, 0:1]
        top1 = idx_ref[:, 1:2]
        iota = lax.broadcasted_iota(jnp.int32, (N_TOK, N_EXP), 1)
        p0 = jnp.sum(jnp.where(iota == top0, probs, 0.0), axis=-1, keepdims=True)
        p1 = jnp.sum(jnp.where(iota == top1, probs, 0.0), axis=-1, keepdims=True)
        gsum = p0 + p1

        xb = xf.astype(jnp.bfloat16)
        acc = jnp.zeros((N_TOK, D_OUT), jnp.float32)
        for j in range(EXP_PER_DEV):
            g = my * EXP_PER_DEV + j
            routed = jnp.logical_or(top0 == g, top1 == g)
            pg = jnp.sum(jnp.where(iota == g, probs, 0.0), axis=-1, keepdims=True)
            w = jnp.where(routed, pg / gsum, 0.0)
            y = jnp.dot(
                xb, ew_ref[j, :, :].astype(jnp.bfloat16),
                preferred_element_type=jnp.float32,
            )
            acc = acc + w * y

        out_ref[:, :] = acc
        sbs = (sb0, sb1, sb2)
        rbs = (rb0, rb1, rb2)

        def start_exchange(s, r):
            lo, hi = STRIPES[s]
            sbs[s][r, :, :] = out_ref[lo:hi, :].astype(jnp.bfloat16)
            rdma = pltpu.make_async_remote_copy(
                src_ref=sbs[s].at[r],
                dst_ref=rbs[s].at[r],
                send_sem=send_sems.at[s, r],
                recv_sem=recv_sems.at[s, r],
                device_id=(partners[(s + r) % 3],),
                device_id_type=pl.DeviceIdType.MESH,
            )
            rdma.start()
            return rdma

        rdmas = {}
        for s in range(3):
            rdmas[(s, 0)] = start_exchange(s, 0)
        for r in range(3):
            for s in range(3):
                rdmas[(s, r)].wait()
                lo, hi = STRIPES[s]
                out_ref[lo:hi, :] = (
                    out_ref[lo:hi, :] + rbs[s][r, :, :].astype(jnp.float32)
                )
                if r < 2:
                    rdmas[(s, r + 1)] = start_exchange(s, r + 1)

    return pl.pallas_call(
        body,
        out_shape=jax.ShapeDtypeStruct((N_TOK, D_OUT), jnp.float32),
        in_specs=[
            pl.BlockSpec(memory_space=pltpu.VMEM),
            pl.BlockSpec(memory_space=pltpu.VMEM),
            pl.BlockSpec(memory_space=pltpu.VMEM),
            pl.BlockSpec(memory_space=pltpu.VMEM),
        ],
        out_specs=pl.BlockSpec(memory_space=pltpu.VMEM),
        scratch_shapes=[
            pltpu.VMEM((3, 176, D_OUT), jnp.bfloat16),
            pltpu.VMEM((3, 168, D_OUT), jnp.bfloat16),
            pltpu.VMEM((3, 168, D_OUT), jnp.bfloat16),
            pltpu.VMEM((3, 176, D_OUT), jnp.bfloat16),
            pltpu.VMEM((3, 168, D_OUT), jnp.bfloat16),
            pltpu.VMEM((3, 168, D_OUT), jnp.bfloat16),
            pltpu.SemaphoreType.DMA((3, 3)),
            pltpu.SemaphoreType.DMA((3, 3)),
        ],
        compiler_params=pltpu.CompilerParams(collective_id=0),
    )(x, router_W, route_idx, expert_W)


# device time: 20367 ns/iter; 3.0306x vs baseline; 1.0072x over previous
import functools

import jax
import jax.numpy as jnp
from jax import lax
from jax.experimental import pallas as pl
from jax.experimental.pallas import tpu as pltpu

N_DEV = 8
N_TOK = 512
D_IN = 256
D_OUT = 512
N_EXP = 16
EXP_PER_DEV = N_EXP // N_DEV


def kernel(x, router_W, route_idx, expert_W):
    XOR_MASKS = (1, 3, 4)
    STRIPES = ((0, 176), (176, 344), (344, 512))

    def body(x_ref, rw_ref, idx_ref, ew_ref, out_ref,
             sb0, sb1, sb2, rb0, rb1, rb2, send_sems, recv_sems):
        my = lax.axis_index("i")
        partners = [jnp.bitwise_xor(my, m) for m in XOR_MASKS]

        barrier_sem = pltpu.get_barrier_semaphore()
        for nbr in partners:
            pl.semaphore_signal(
                barrier_sem, inc=1,
                device_id=(nbr,), device_id_type=pl.DeviceIdType.MESH,
            )
        pl.semaphore_wait(barrier_sem, len(partners))

        xf = x_ref[:, :]
        scores = jnp.dot(xf, rw_ref[:, :], preferred_element_type=jnp.float32)
        s_max = jnp.max(scores, axis=-1, keepdims=True)
        ex = jnp.exp(scores - s_max)
        probs = ex / jnp.sum(ex, axis=-1, keepdims=True)

        top0 = idx_ref[:, 0:1]
        top1 = idx_ref[:, 1:2]
        iota = lax.broadcasted_iota(jnp.int32, (N_TOK, N_EXP), 1)
        p0 = jnp.sum(jnp.where(iota == top0, probs, 0.0), axis=-1, keepdims=True)
        p1 = jnp.sum(jnp.where(iota == top1, probs, 0.0), axis=-1, keepdims=True)
        gsum = p0 + p1

        ws = []
        for j in range(EXP_PER_DEV):
            g = my * EXP_PER_DEV + j
            routed = jnp.logical_or(top0 == g, top1 == g)
            pg = jnp.sum(jnp.where(iota == g, probs, 0.0), axis=-1, keepdims=True)
            ws.append(jnp.where(routed, pg / gsum, 0.0))
        xb = xf.astype(jnp.bfloat16)
        wb = [ew_ref[j, :, :].astype(jnp.bfloat16) for j in range(EXP_PER_DEV)]

        sbs = (sb0, sb1, sb2)
        rbs = (rb0, rb1, rb2)

        def start_exchange(s, r, val_bf16):
            sbs[s][r, :, :] = val_bf16
            rdma = pltpu.make_async_remote_copy(
                src_ref=sbs[s].at[r],
                dst_ref=rbs[s].at[r],
                send_sem=send_sems.at[s, r],
                recv_sem=recv_sems.at[s, r],
                device_id=(partners[(s + r) % 3],),
                device_id_type=pl.DeviceIdType.MESH,
            )
            rdma.start()
            return rdma

        rdmas = {}
        accs = [None, None, None]
        for s in range(3):
            lo, hi = STRIPES[s]
            a = jnp.zeros((hi - lo, D_OUT), jnp.float32)
            for j in range(EXP_PER_DEV):
                y = jnp.dot(xb[lo:hi, :], wb[j], preferred_element_type=jnp.float32)
                a = a + ws[j][lo:hi, :] * y
            accs[s] = a
            rdmas[(s, 0)] = start_exchange(s, 0, a.astype(jnp.bfloat16))
        for r in range(3):
            for s in range(3):
                rdmas[(s, r)].wait()
                accs[s] = accs[s] + rbs[s][r, :, :].astype(jnp.float32)
                if r < 2:
                    rdmas[(s, r + 1)] = start_exchange(
                        s, r + 1, accs[s].astype(jnp.bfloat16)
                    )
                else:
                    lo, hi = STRIPES[s]
                    out_ref[lo:hi, :] = accs[s]

    return pl.pallas_call(
        body,
        out_shape=jax.ShapeDtypeStruct((N_TOK, D_OUT), jnp.float32),
        in_specs=[
            pl.BlockSpec(memory_space=pltpu.VMEM),
            pl.BlockSpec(memory_space=pltpu.VMEM),
            pl.BlockSpec(memory_space=pltpu.VMEM),
            pl.BlockSpec(memory_space=pltpu.VMEM),
        ],
        out_specs=pl.BlockSpec(memory_space=pltpu.VMEM),
        scratch_shapes=[
            pltpu.VMEM((3, 176, D_OUT), jnp.bfloat16),
            pltpu.VMEM((3, 168, D_OUT), jnp.bfloat16),
            pltpu.VMEM((3, 168, D_OUT), jnp.bfloat16),
            pltpu.VMEM((3, 176, D_OUT), jnp.bfloat16),
            pltpu.VMEM((3, 168, D_OUT), jnp.bfloat16),
            pltpu.VMEM((3, 168, D_OUT), jnp.bfloat16),
            pltpu.SemaphoreType.DMA((3, 3)),
            pltpu.SemaphoreType.DMA((3, 3)),
        ],
        compiler_params=pltpu.CompilerParams(collective_id=0),
    )(x, router_W, route_idx, expert_W)


# device time: 20302 ns/iter; 3.0403x vs baseline; 1.0032x over previous
import functools

import jax
import jax.numpy as jnp
from jax import lax
from jax.experimental import pallas as pl
from jax.experimental.pallas import tpu as pltpu

N_DEV = 8
N_TOK = 512
D_IN = 256
D_OUT = 512
N_EXP = 16
EXP_PER_DEV = N_EXP // N_DEV


def kernel(x, router_W, route_idx, expert_W):
    XOR_MASKS = (1, 3, 4)
    STRIPES = ((0, 176), (176, 344), (344, 512))

    def body(x_ref, rw_ref, idx_ref, ew_ref, out_ref,
             sb0, sb1, sb2, rb0, rb1, rb2, send_sems, recv_sems):
        my = lax.axis_index("i")
        partners = [jnp.bitwise_xor(my, m) for m in XOR_MASKS]

        barrier_sem = pltpu.get_barrier_semaphore()
        for nbr in partners:
            pl.semaphore_signal(
                barrier_sem, inc=1,
                device_id=(nbr,), device_id_type=pl.DeviceIdType.MESH,
            )
        pl.semaphore_wait(barrier_sem, len(partners))

        xf = x_ref[:, :]
        scores = jnp.dot(xf, rw_ref[:, :], preferred_element_type=jnp.float32)
        s_max = jnp.max(scores, axis=-1, keepdims=True)
        ex = jnp.exp(scores - s_max)
        probs = ex / jnp.sum(ex, axis=-1, keepdims=True)

        top0 = idx_ref[:, 0:1]
        top1 = idx_ref[:, 1:2]
        iota = lax.broadcasted_iota(jnp.int32, (N_TOK, N_EXP), 1)
        p0 = jnp.sum(jnp.where(iota == top0, probs, 0.0), axis=-1, keepdims=True)
        p1 = jnp.sum(jnp.where(iota == top1, probs, 0.0), axis=-1, keepdims=True)
        gsum = p0 + p1

        ws = []
        for j in range(EXP_PER_DEV):
            g = my * EXP_PER_DEV + j
            routed = jnp.logical_or(top0 == g, top1 == g)
            pg = jnp.sum(jnp.where(iota == g, probs, 0.0), axis=-1, keepdims=True)
            ws.append(jnp.where(routed, pg / gsum, 0.0))
        xb = xf.astype(jnp.bfloat16)
        wb = [ew_ref[j, :, :].astype(jnp.bfloat16) for j in range(EXP_PER_DEV)]

        sbs = (sb0, sb1, sb2)
        rbs = (rb0, rb1, rb2)

        def start_exchange(s, r, val_bf16):
            sbs[s][r, :, :] = val_bf16
            rdma = pltpu.make_async_remote_copy(
                src_ref=sbs[s].at[r],
                dst_ref=rbs[s].at[r],
                send_sem=send_sems.at[s, r],
                recv_sem=recv_sems.at[s, r],
                device_id=(partners[(s + r) % 3],),
                device_id_type=pl.DeviceIdType.MESH,
            )
            rdma.start()
            return rdma

        N_ROUNDS = 3
        rdmas = {}
        accs = [None, None, None]
        for s in range(3):
            lo, hi = STRIPES[s]
            a = jnp.zeros((hi - lo, D_OUT), jnp.float32)
            for j in range(EXP_PER_DEV):
                y = jnp.dot(xb[lo:hi, :], wb[j], preferred_element_type=jnp.float32)
                a = a + ws[j][lo:hi, :] * y
            accs[s] = a
            if N_ROUNDS > 0:
                rdmas[(s, 0)] = start_exchange(s, 0, a.astype(jnp.bfloat16))
            else:
                out_ref[lo:hi, :] = a
        for r in range(N_ROUNDS):
            for s in range(3):
                rdmas[(s, r)].wait()
                accs[s] = accs[s] + rbs[s][r, :, :].astype(jnp.float32)
                if r < N_ROUNDS - 1:
                    rdmas[(s, r + 1)] = start_exchange(
                        s, r + 1, accs[s].astype(jnp.bfloat16)
                    )
                else:
                    lo, hi = STRIPES[s]
                    out_ref[lo:hi, :] = accs[s]

    return pl.pallas_call(
        body,
        out_shape=jax.ShapeDtypeStruct((N_TOK, D_OUT), jnp.float32),
        in_specs=[
            pl.BlockSpec(memory_space=pltpu.VMEM),
            pl.BlockSpec(memory_space=pltpu.VMEM),
            pl.BlockSpec(memory_space=pltpu.VMEM),
            pl.BlockSpec(memory_space=pltpu.VMEM),
        ],
        out_specs=pl.BlockSpec(memory_space=pltpu.VMEM),
        scratch_shapes=[
            pltpu.VMEM((3, 176, D_OUT), jnp.bfloat16),
            pltpu.VMEM((3, 168, D_OUT), jnp.bfloat16),
            pltpu.VMEM((3, 168, D_OUT), jnp.bfloat16),
            pltpu.VMEM((3, 176, D_OUT), jnp.bfloat16),
            pltpu.VMEM((3, 168, D_OUT), jnp.bfloat16),
            pltpu.VMEM((3, 168, D_OUT), jnp.bfloat16),
            pltpu.SemaphoreType.DMA((3, 3)),
            pltpu.SemaphoreType.DMA((3, 3)),
        ],
        compiler_params=pltpu.CompilerParams(collective_id=0),
    )(x, router_W, route_idx, expert_W)
